# baseline (speedup 1.0000x reference)
import jax
import jax.numpy as jnp
from jax import lax
from jax.experimental import pallas as pl
from jax.experimental.pallas import tpu as pltpu
from jax.experimental.pallas import tpu_sc as plsc

EMB = 64
EMBP = 128
BATCH = 4096
HIST = 50
VOCAB = 1000000
ZROW = VOCAB
VOCABP = VOCAB + 8

NW = 32
BPW = BATCH // NW
NB = 8
NCHUNK = BPW // NB


def _sc_body(
    table_hbm, idx_hbm, out_hbm,
    idx_a, idx_b, rows_a, rows_b, gsem_a, gsem_b, wsem_a, wsem_b,
):
    wid = lax.axis_index("s") * 2 + lax.axis_index("c")
    base = wid * BPW

    idxs = (idx_a, idx_b)
    bufs = (rows_a, rows_b)
    gsems = (gsem_a, gsem_b)
    wsems = (wsem_a, wsem_b)

    def stage_idx(i):
        pltpu.sync_copy(idx_hbm.at[pl.ds(base + i * NB, NB)], idxs[i % 2])

    def fire_gathers(i):
        iv, buf, sem = idxs[i % 2], bufs[i % 2], gsems[i % 2]
        return [
            pltpu.async_copy(table_hbm.at[iv.at[jb]], buf.at[jb], sem)
            for jb in range(NB)
        ]

    flush_desc = [None, None]
    stage_idx(0)
    g_descs = fire_gathers(0)
    for i in range(NCHUNK):
        b = i % 2
        nxt = None
        if i + 1 < NCHUNK:
            nb = (i + 1) % 2
            if flush_desc[nb] is not None:
                flush_desc[nb].wait()
                flush_desc[nb] = None
            stage_idx(i + 1)
            nxt = fire_gathers(i + 1)
        for d in g_descs:
            d.wait()
        flush_desc[b] = pltpu.async_copy(
            bufs[b], out_hbm.at[pl.ds(base + i * NB, NB)], wsems[b]
        )
        g_descs = nxt
    for fd in flush_desc:
        if fd is not None:
            fd.wait()


def kernel(inputs, table):
    idx = inputs.astype(jnp.int32)
    idx = jnp.where(idx == 0, ZROW, idx)
    tpad = jnp.pad(table, ((0, VOCABP - VOCAB), (0, EMBP - EMB)))
    mesh = plsc.VectorSubcoreMesh(core_axis_name="c", subcore_axis_name="s")
    k = pl.kernel(
        _sc_body,
        out_type=jax.ShapeDtypeStruct((BATCH, HIST, EMBP), jnp.float32),
        mesh=mesh,
        scratch_types=[
            pltpu.VMEM((NB, HIST), jnp.int32),
            pltpu.VMEM((NB, HIST), jnp.int32),
            pltpu.VMEM((NB, HIST, EMBP), jnp.float32),
            pltpu.VMEM((NB, HIST, EMBP), jnp.float32),
            pltpu.SemaphoreType.DMA,
            pltpu.SemaphoreType.DMA,
            pltpu.SemaphoreType.DMA,
            pltpu.SemaphoreType.DMA,
        ],
        compiler_params=pltpu.CompilerParams(
            use_tc_tiling_on_sc=True, needs_layout_passes=False
        ),
    )
    out = k(tpad, idx)
    return out[:, :, :EMB]
